# baseline (speedup 1.0000x reference)
import functools

import jax
import jax.numpy as jnp
from jax import lax
from jax.experimental import pallas as pl
from jax.experimental.pallas import tpu as pltpu
from jax.experimental.pallas import tpu_sc as plsc

N = 10000
E = 160000
H = 4
D = 128
NT = D // 16

NC, NS = 2, 16
NW = NC * NS
NPS = 320
RPP = NPS * NW + 336
BLK = 128
OUTW = H * D
FLROWS = 16
FLW = FLROWS * OUTW


def _celu3(x):
    return jnp.where(x > 0, x, 3.0 * jnp.exp(x * (1.0 / 3.0)) - 3.0)


def _a1_body(f_ref, w_ref, o_ref):
    x = jnp.dot(f_ref[...], w_ref[...], preferred_element_type=jnp.float32)
    o_ref[...] = _celu3(x)


def _a1_tc(features, w16):
    return pl.pallas_call(
        _a1_body,
        grid=(10,),
        in_specs=[
            pl.BlockSpec((N // 10, D), lambda i: (i, 0)),
            pl.BlockSpec((D, 16), lambda i: (0, 0)),
        ],
        out_specs=pl.BlockSpec((N // 10, 16), lambda i: (i, 0)),
        out_shape=jax.ShapeDtypeStruct((N, 16), jnp.float32),
    )(features, w16)


_MESH = plsc.VectorSubcoreMesh(
    core_axis_name="c", subcore_axis_name="s", num_cores=NC, num_subcores=NS
)


@functools.partial(
    pl.kernel,
    out_type=jax.ShapeDtypeStruct((N * OUTW,), jnp.float32),
    mesh=_MESH,
    compiler_params=pltpu.CompilerParams(needs_layout_passes=False),
    scratch_types=[
        pltpu.VMEM((336,), jnp.int32),
        pltpu.VMEM((NPS * 16,), jnp.float32),
        pltpu.VMEM((BLK * D,), jnp.float32),
        pltpu.VMEM((H * D,), jnp.float32),
        pltpu.VMEM((2 * FLW,), jnp.float32),
        pltpu.SemaphoreType.DMA,
        pltpu.SemaphoreType.DMA,
    ],
)
def _sc_main(rowptr_hbm, a1_hbm, emb_hbm, attn2_hbm, out_hbm,
             rowptr_v, a1_v, emb_v, attn2_v, stage_v, sem0, sem1):
    wid = lax.axis_index("c") * NS + lax.axis_index("s")
    n0 = wid * NPS
    num = jnp.minimum(NPS, N - n0)

    pltpu.sync_copy(rowptr_hbm.at[pl.ds(pl.multiple_of(n0, 16), 336)], rowptr_v)
    pltpu.sync_copy(a1_hbm.at[pl.ds(pl.multiple_of(n0 * 16, 16), NPS * 16)], a1_v)
    pltpu.sync_copy(attn2_hbm, attn2_v)

    lanes = lax.iota(jnp.int32, 16)
    oneh = [(lanes == h).astype(jnp.float32) for h in range(H)]
    idxc = [jnp.full((16,), h, jnp.int32) for h in range(H)]

    def rp(j):
        base = pl.multiple_of((j // 16) * 16, 16)
        vec = rowptr_v[pl.ds(base, 16)]
        return jnp.sum(jnp.where(lanes == (j - base), vec, 0))

    _dn = lax.GatherDimensionNumbers(
        offset_dims=(), collapsed_slice_dims=(0,), start_index_map=(0,))

    def bc(v, h):
        return lax.gather(
            v, idxc[h][:, None], dimension_numbers=_dn, slice_sizes=(1,),
            mode=lax.GatherScatterMode.PROMISE_IN_BOUNDS)

    def node_body(n_loc, base):
        start = rp(n_loc)
        end = rp(n_loc + 1)
        a1row = a1_v[pl.ds(pl.multiple_of(n_loc * 16, 16), 16)]

        def inst_body(r, carry):
            base, m, d, o = carry

            def refill():
                s = jnp.minimum(r, E - BLK)
                pltpu.sync_copy(
                    emb_hbm.at[pl.ds(pl.multiple_of(s * D, D), BLK * D)], emb_v)
                return s

            base = lax.cond(r >= base + BLK, refill, lambda: base)
            roff = pl.multiple_of((r - base) * D, D)

            eft = [_celu3(emb_v[pl.ds(roff + 16 * t, 16)]) for t in range(NT)]
            a2vec = jnp.zeros((16,), jnp.float32)
            for h in range(H):
                acc = eft[0] * attn2_v[pl.ds(h * D, 16)]
                for t in range(1, NT):
                    acc = acc + eft[t] * attn2_v[pl.ds(h * D + 16 * t, 16)]
                a2vec = a2vec + jnp.sum(acc) * oneh[h]

            av = _celu3(a1row + a2vec)
            mn = jnp.maximum(m, av)
            sc = jnp.exp(m - mn)
            w = jnp.exp(av - mn)
            d = d * sc + w
            o = list(o)
            for h in range(H):
                wh = bc(w, h)
                sch = bc(sc, h)
                for t in range(NT):
                    o[h * NT + t] = o[h * NT + t] * sch + wh * eft[t]
            return base, mn, d, tuple(o)
        m0 = jnp.full((16,), -1e30, jnp.float32)
        d0 = jnp.zeros((16,), jnp.float32)
        o0 = tuple(jnp.zeros((16,), jnp.float32) for _ in range(H * NT))
        base, m, d, o = lax.fori_loop(start, end, inst_body, (base, m0, d0, o0))

        f = lax.shift_right_logical(n_loc, 4)
        within = jnp.bitwise_and(n_loc, 15)
        parity = jnp.bitwise_and(f, 1)

        def wait_par():
            def w0():
                pltpu.make_async_copy(
                    stage_v.at[pl.ds(0, FLW)], out_hbm.at[pl.ds(0, FLW)], sem0
                ).wait()
            def w1():
                pltpu.make_async_copy(
                    stage_v.at[pl.ds(FLW, FLW)], out_hbm.at[pl.ds(0, FLW)], sem1
                ).wait()
            lax.cond(parity == 0, w0, w1)

        lax.cond(jnp.logical_and(within == 0, f >= 2), wait_par, lambda: None)

        st = pl.multiple_of(parity * FLW + within * OUTW, OUTW)
        for h in range(H):
            dh = bc(d, h)
            inv = 1.0 / jnp.where(dh > 0, dh, 1.0)
            for t in range(NT):
                stage_v[pl.ds(st + h * D + 16 * t, 16)] = _celu3(
                    o[h * NT + t] * inv)

        def start_flush():
            dst = pl.multiple_of((n0 + n_loc - (FLROWS - 1)) * OUTW, FLW)
            def s0():
                pltpu.async_copy(
                    stage_v.at[pl.ds(0, FLW)], out_hbm.at[pl.ds(dst, FLW)], sem0)
            def s1():
                pltpu.async_copy(
                    stage_v.at[pl.ds(FLW, FLW)], out_hbm.at[pl.ds(dst, FLW)], sem1)
            lax.cond(parity == 0, s0, s1)

        lax.cond(within == FLROWS - 1, start_flush, lambda: None)
        return base

    lax.fori_loop(0, num, node_body, jnp.int32(-2 * BLK))

    pltpu.make_async_copy(
        stage_v.at[pl.ds(0, FLW)], out_hbm.at[pl.ds(0, FLW)], sem0).wait()
    pltpu.make_async_copy(
        stage_v.at[pl.ds(FLW, FLW)], out_hbm.at[pl.ds(0, FLW)], sem1).wait()


def kernel(nodes, metapath_instances, metapath_embedding, features, attn1_w, attn2):
    seg = metapath_instances
    rowptr = jnp.searchsorted(seg, jnp.arange(N + 1, dtype=jnp.int32)).astype(jnp.int32)
    rowptr_pad = jnp.concatenate(
        [rowptr, jnp.full((RPP - (N + 1),), E, jnp.int32)])
    w16 = jnp.zeros((D, 16), jnp.float32).at[:, :H].set(attn1_w.T)
    a1 = _a1_tc(features, w16)
    out = _sc_main(
        rowptr_pad,
        a1.reshape(-1),
        metapath_embedding.reshape(-1),
        attn2.reshape(-1),
    )
    return out.reshape(N, OUTW)


# speedup vs baseline: 6.9176x; 6.9176x over previous
import functools

import jax
import jax.numpy as jnp
from jax import lax
from jax.experimental import pallas as pl
from jax.experimental.pallas import tpu as pltpu
from jax.experimental.pallas import tpu_sc as plsc

N = 10000
E = 160000
H = 4
D = 128
NT = D // 16

NC, NS = 2, 16
NW = NC * NS
NPS = 320
BLK = 128
OUTW = H * D
FLROWS = 16
FLW = FLROWS * OUTW


def _celu3(x):
    return jnp.where(x > 0, x, 3.0 * jnp.exp(x * (1.0 / 3.0)) - 3.0)


def _a1_body(f_ref, w_ref, o_ref):
    x = jnp.dot(f_ref[...], w_ref[...], preferred_element_type=jnp.float32)
    o_ref[...] = _celu3(x)


def _a1_tc(features, w16):
    return pl.pallas_call(
        _a1_body,
        grid=(10,),
        in_specs=[
            pl.BlockSpec((N // 10, D), lambda i: (i, 0)),
            pl.BlockSpec((D, 16), lambda i: (0, 0)),
        ],
        out_specs=pl.BlockSpec((N // 10, 16), lambda i: (i, 0)),
        out_shape=jax.ShapeDtypeStruct((N, 16), jnp.float32),
    )(features, w16)


_MESH = plsc.VectorSubcoreMesh(
    core_axis_name="c", subcore_axis_name="s", num_cores=NC, num_subcores=NS
)


@functools.partial(
    pl.kernel,
    out_type=jax.ShapeDtypeStruct((N * OUTW,), jnp.float32),
    mesh=_MESH,
    compiler_params=pltpu.CompilerParams(needs_layout_passes=False),
    scratch_types=[
        pltpu.VMEM((NPS * 16,), jnp.float32),
        pltpu.VMEM((2 * BLK,), jnp.int32),
        pltpu.VMEM((2 * BLK * D,), jnp.float32),
        pltpu.VMEM((H * D,), jnp.float32),
        pltpu.VMEM((2 * FLW,), jnp.float32),
        pltpu.VMEM((16,), jnp.int32),
        pltpu.SemaphoreType.DMA,
        pltpu.SemaphoreType.DMA,
        pltpu.SemaphoreType.DMA,
        pltpu.SemaphoreType.DMA,
        pltpu.SemaphoreType.DMA,
        pltpu.SemaphoreType.DMA,
    ],
)
def _sc_main(a1_hbm, seg_hbm, emb_hbm, attn2_hbm, out_hbm,
             a1_v, seg_v, emb_v, attn2_v, stage_v, probe_v,
             semE0, semE1, semS0, semS1, semF0, semF1):
    wid = lax.axis_index("c") * NS + lax.axis_index("s")
    n0 = wid * NPS
    num = jnp.minimum(NPS, N - n0)
    n1 = n0 + num

    pltpu.sync_copy(a1_hbm.at[pl.ds(pl.multiple_of(n0 * 16, 16), NPS * 16)], a1_v)
    pltpu.sync_copy(attn2_hbm.at[pl.ds(0, H * D)], attn2_v)

    lanes = lax.iota(jnp.int32, 16)
    oneh = [(lanes == h).astype(jnp.float32) for h in range(H)]
    idxc = [jnp.full((16,), h, jnp.int32) for h in range(H)]
    _dn = lax.GatherDimensionNumbers(
        offset_dims=(), collapsed_slice_dims=(0,), start_index_map=(0,))
    zero16 = jnp.zeros((16,), jnp.float32)
    d0 = zero16
    o0 = tuple(zero16 for _ in range(H * NT))
    m0 = jnp.full((16,), -1e30, jnp.float32)

    def bc(v, h):
        return lax.gather(v, idxc[h][:, None], dimension_numbers=_dn,
                          slice_sizes=(1,),
                          mode=lax.GatherScatterMode.PROMISE_IN_BOUNDS)

    def ext_i32(vec, lane):
        return jnp.sum(jnp.where(lanes == lane, vec, 0))

    def seg_probe(pos):
        off = jnp.minimum(pos - lax.rem(pos, 8), E - 16)
        off = pl.multiple_of(off, 8)
        pltpu.sync_copy(seg_hbm.at[pl.ds(off, 16)], probe_v)
        return ext_i32(probe_v[pl.ds(0, 16)], pos - off)

    def lower_bound(tgt):
        def b(_, st):
            lo, hi = st
            mid = (lo + hi) // 2
            v = seg_probe(mid)
            act = lo < hi
            big = v >= tgt
            lo2 = jnp.where(jnp.logical_and(act, jnp.logical_not(big)), mid + 1, lo)
            hi2 = jnp.where(jnp.logical_and(act, big), mid, hi)
            return lo2, hi2
        lo, _ = lax.fori_loop(0, 18, b, (jnp.int32(0), jnp.int32(E)))
        return lo

    r0 = lower_bound(n0)
    r1 = jnp.where(n1 >= N, jnp.int32(E), lower_bound(n1))

    def emit(n_loc, d, o):
        f = lax.shift_right_logical(n_loc, 4)
        within = jnp.bitwise_and(n_loc, 15)
        parity = jnp.bitwise_and(f, 1)

        def wait_par():
            def w0():
                pltpu.make_async_copy(
                    stage_v.at[pl.ds(0, FLW)], out_hbm.at[pl.ds(0, FLW)], semF0
                ).wait()
            def w1():
                pltpu.make_async_copy(
                    stage_v.at[pl.ds(FLW, FLW)], out_hbm.at[pl.ds(0, FLW)], semF1
                ).wait()
            lax.cond(parity == 0, w0, w1)

        lax.cond(jnp.logical_and(within == 0, f >= 2), wait_par, lambda: None)

        st = pl.multiple_of(parity * FLW + within * OUTW, OUTW)
        for h in range(H):
            dh = bc(d, h)
            inv = 1.0 / jnp.where(dh > 0, dh, 1.0)
            for t in range(NT):
                stage_v[pl.ds(st + h * D + 16 * t, 16)] = _celu3(
                    o[h * NT + t] * inv)

        def start_flush():
            dst = pl.multiple_of((n0 + n_loc - (FLROWS - 1)) * OUTW, FLW)
            def s0():
                pltpu.async_copy(
                    stage_v.at[pl.ds(0, FLW)], out_hbm.at[pl.ds(dst, FLW)], semF0)
            def s1():
                pltpu.async_copy(
                    stage_v.at[pl.ds(FLW, FLW)], out_hbm.at[pl.ds(dst, FLW)], semF1)
            lax.cond(parity == 0, s0, s1)

        lax.cond(within == FLROWS - 1, start_flush, lambda: None)

    def emit_zero(n_loc):
        emit(n_loc, d0, o0)

    BS0 = (r0 // BLK) * BLK
    nblocks = (r1 - BS0 + BLK - 1) // BLK

    def start_block(gi):
        bs = BS0 + gi * BLK
        slot = jnp.bitwise_and(gi, 1)
        def s0():
            pltpu.async_copy(emb_hbm.at[pl.ds(pl.multiple_of(bs * D, D), BLK * D)],
                             emb_v.at[pl.ds(0, BLK * D)], semE0)
            pltpu.async_copy(seg_hbm.at[pl.ds(pl.multiple_of(bs, 8), BLK)],
                             seg_v.at[pl.ds(0, BLK)], semS0)
        def s1():
            pltpu.async_copy(emb_hbm.at[pl.ds(pl.multiple_of(bs * D, D), BLK * D)],
                             emb_v.at[pl.ds(BLK * D, BLK * D)], semE1)
            pltpu.async_copy(seg_hbm.at[pl.ds(pl.multiple_of(bs, 8), BLK)],
                             seg_v.at[pl.ds(BLK, BLK)], semS1)
        lax.cond(slot == 0, s0, s1)

    def wait_block(parity):
        def w0():
            pltpu.make_async_copy(emb_hbm.at[pl.ds(0, BLK * D)],
                                  emb_v.at[pl.ds(0, BLK * D)], semE0).wait()
            pltpu.make_async_copy(seg_hbm.at[pl.ds(0, BLK)],
                                  seg_v.at[pl.ds(0, BLK)], semS0).wait()
        def w1():
            pltpu.make_async_copy(emb_hbm.at[pl.ds(0, BLK * D)],
                                  emb_v.at[pl.ds(BLK * D, BLK * D)], semE1).wait()
            pltpu.make_async_copy(seg_hbm.at[pl.ds(0, BLK)],
                                  seg_v.at[pl.ds(BLK, BLK)], semS1).wait()
        lax.cond(parity == 0, w0, w1)

    n_init = jnp.where(r0 < r1, seg_probe(jnp.minimum(r0, E - 1)), n0)

    lax.fori_loop(n0, n_init, lambda k, _: (emit_zero(k - n0), 0)[1], 0)

    lax.cond(nblocks > 0, lambda: start_block(jnp.int32(0)), lambda: None)

    def block_body(g, carry):
        p_in, n_in, m_in, dv_in, o_in = carry
        parity = jnp.bitwise_and(g, 1)
        wait_block(parity)
        lax.cond(g + 1 < nblocks, lambda: start_block(g + 1), lambda: None)
        bs = BS0 + g * BLK
        be = jnp.minimum(bs + BLK, r1)
        vbase = parity * (BLK * D)
        sbase = parity * BLK

        def seg_at(idx_local):
            grp = pl.multiple_of(sbase + (idx_local // 16) * 16, 16)
            return ext_i32(seg_v[pl.ds(grp, 16)], idx_local - (idx_local // 16) * 16)

        def walk_cond(st):
            return st[0] < be

        def walk_body(st):
            p, n, m, dv, o = st
            a1row = a1_v[pl.ds(pl.multiple_of((n - n0) * 16, 16), 16)]

            def scan_cond(s2):
                q, e = s2
                return jnp.logical_and(e < 0, q < be - bs)

            def scan_body(s2):
                q, _ = s2
                qa = pl.multiple_of(sbase + q, 16)
                v = seg_v[pl.ds(qa, 16)]
                inwin = jnp.logical_and(q + lanes >= p - bs, q + lanes < be - bs)
                vm = jnp.where(inwin, v, n)
                fs = jnp.max(plsc.all_reduce_ffs(vm > n))
                e2 = jnp.where(fs < 16, q + fs, -1)
                return q + 16, e2

            q0 = ((p - bs) // 16) * 16
            _, e_loc = lax.while_loop(scan_cond, scan_body, (q0, jnp.int32(-1)))
            e_abs = bs + jnp.where(e_loc >= 0, e_loc, be - bs)

            def inst_body(r, c3):
                m, dv, o = c3
                roff = pl.multiple_of(vbase + (r - bs) * D, 16)
                eft = [_celu3(emb_v[pl.ds(roff + 16 * t, 16)]) for t in range(NT)]
                a2vec = zero16
                for h in range(H):
                    acc = eft[0] * attn2_v[pl.ds(h * D, 16)]
                    for t in range(1, NT):
                        acc = acc + eft[t] * attn2_v[pl.ds(h * D + 16 * t, 16)]
                    a2vec = a2vec + jnp.sum(acc) * oneh[h]
                av = _celu3(a1row + a2vec)
                mn = jnp.maximum(m, av)
                sc = jnp.exp(m - mn)
                w = jnp.exp(av - mn)
                dv = dv * sc + w
                o = list(o)
                for h in range(H):
                    wh = bc(w, h)
                    sch = bc(sc, h)
                    for t in range(NT):
                        o[h * NT + t] = o[h * NT + t] * sch + wh * eft[t]
                return mn, dv, tuple(o)

            m, dv, o = lax.fori_loop(p, e_abs, inst_body, (m, dv, o))

            def fin():
                emit(n - n0, dv, o)
                nn = seg_at(e_abs - bs)
                lax.fori_loop(n + 1, nn,
                              lambda k, _: (emit_zero(k - n0), 0)[1], 0)
                return nn, m0, d0, o0

            def keep():
                return n, m, dv, o

            n2, m2, dv2, o2 = lax.cond(e_abs < be, fin, keep)
            return e_abs, n2, m2, dv2, o2

        p_out, n_out, m_out, dv_out, o_out = lax.while_loop(
            walk_cond, walk_body, (jnp.maximum(p_in, bs), n_in, m_in, dv_in, o_in))
        return p_out, n_out, m_out, dv_out, o_out

    _, n_fin, _, dv_fin, o_fin = lax.fori_loop(
        0, nblocks, block_body, (r0, n_init, m0, d0, o0))

    emit(n_fin - n0, dv_fin, o_fin)
    lax.fori_loop(n_fin + 1, n1, lambda k, _: (emit_zero(k - n0), 0)[1], 0)

    pltpu.make_async_copy(
        stage_v.at[pl.ds(0, FLW)], out_hbm.at[pl.ds(0, FLW)], semF0).wait()
    pltpu.make_async_copy(
        stage_v.at[pl.ds(FLW, FLW)], out_hbm.at[pl.ds(0, FLW)], semF1).wait()


def kernel(nodes, metapath_instances, metapath_embedding, features, attn1_w, attn2):
    w16 = jnp.zeros((D, 16), jnp.float32).at[:, :H].set(attn1_w.T)
    a1 = _a1_tc(features, w16)
    out = _sc_main(
        a1.reshape(-1),
        metapath_instances,
        metapath_embedding.reshape(-1),
        attn2.reshape(-1),
    )
    return out.reshape(N, OUTW)


# speedup vs baseline: 7.9664x; 1.1516x over previous
import functools

import jax
import jax.numpy as jnp
from jax import lax
from jax.experimental import pallas as pl
from jax.experimental.pallas import tpu as pltpu
from jax.experimental.pallas import tpu_sc as plsc

N = 10000
E = 160000
H = 4
D = 128
NT = D // 16

NC, NS = 2, 16
NW = NC * NS
NPS = 320
BLK = 128
OUTW = H * D
FLROWS = 16
FLW = FLROWS * OUTW


def _celu3(x):
    return jnp.where(x > 0, x, 3.0 * jnp.exp(x * (1.0 / 3.0)) - 3.0)


def _a1_body(f_ref, w_ref, o_ref):
    x = jnp.dot(f_ref[...], w_ref[...], preferred_element_type=jnp.float32)
    o_ref[...] = _celu3(x)


def _a1_tc(features, w16):
    return pl.pallas_call(
        _a1_body,
        grid=(10,),
        in_specs=[
            pl.BlockSpec((N // 10, D), lambda i: (i, 0)),
            pl.BlockSpec((D, 16), lambda i: (0, 0)),
        ],
        out_specs=pl.BlockSpec((N // 10, 16), lambda i: (i, 0)),
        out_shape=jax.ShapeDtypeStruct((N, 16), jnp.float32),
    )(features, w16)


_MESH = plsc.VectorSubcoreMesh(
    core_axis_name="c", subcore_axis_name="s", num_cores=NC, num_subcores=NS
)


@functools.partial(
    pl.kernel,
    out_type=jax.ShapeDtypeStruct((N * OUTW,), jnp.float32),
    mesh=_MESH,
    compiler_params=pltpu.CompilerParams(needs_layout_passes=False),
    scratch_types=[
        pltpu.VMEM((NPS * 16,), jnp.float32),
        pltpu.VMEM((2 * BLK,), jnp.int32),
        pltpu.VMEM((2 * BLK * D,), jnp.float32),
        pltpu.VMEM((H * D,), jnp.float32),
        pltpu.VMEM((2 * FLW,), jnp.float32),
        pltpu.VMEM((OUTW,), jnp.float32),
        pltpu.VMEM((16,), jnp.int32),
        pltpu.SemaphoreType.DMA,
        pltpu.SemaphoreType.DMA,
        pltpu.SemaphoreType.DMA,
        pltpu.SemaphoreType.DMA,
        pltpu.SemaphoreType.DMA,
        pltpu.SemaphoreType.DMA,
    ],
)
def _sc_main(a1_hbm, seg_hbm, emb_hbm, attn2_hbm, out_hbm,
             a1_v, seg_v, emb_v, attn2_v, stage_v, o_buf, probe_v,
             semE0, semE1, semS0, semS1, semF0, semF1):
    wid = lax.axis_index("c") * NS + lax.axis_index("s")
    n0 = wid * NPS
    num = jnp.minimum(NPS, N - n0)
    n1 = n0 + num

    pltpu.sync_copy(a1_hbm.at[pl.ds(pl.multiple_of(n0 * 16, 16), NPS * 16)], a1_v)
    pltpu.sync_copy(attn2_hbm.at[pl.ds(0, H * D)], attn2_v)

    lanes = lax.iota(jnp.int32, 16)
    oneh = [(lanes == h).astype(jnp.float32) for h in range(H)]
    idxc = [jnp.full((16,), h, jnp.int32) for h in range(H)]
    _dn = lax.GatherDimensionNumbers(
        offset_dims=(), collapsed_slice_dims=(0,), start_index_map=(0,))
    zero16 = jnp.zeros((16,), jnp.float32)
    d0 = zero16
    m0 = jnp.full((16,), -1e30, jnp.float32)

    a2w = [attn2_v[pl.ds(h * D + 16 * t, 16)] for h in range(H) for t in range(NT)]
    for i in range(H * NT):
        o_buf[pl.ds(16 * i, 16)] = zero16

    def bc(v, h):
        return lax.gather(v, idxc[h][:, None], dimension_numbers=_dn,
                          slice_sizes=(1,),
                          mode=lax.GatherScatterMode.PROMISE_IN_BOUNDS)

    def ext_i32(vec, lane):
        return jnp.sum(jnp.where(lanes == lane, vec, 0))

    def seg_probe(pos):
        off = jnp.minimum(pos - lax.rem(pos, 8), E - 16)
        off = pl.multiple_of(off, 8)
        pltpu.sync_copy(seg_hbm.at[pl.ds(off, 16)], probe_v)
        return ext_i32(probe_v[pl.ds(0, 16)], pos - off)

    def lower_bound(tgt):
        def b(_, st):
            lo, hi = st
            mid = (lo + hi) // 2
            v = seg_probe(mid)
            act = lo < hi
            big = v >= tgt
            lo2 = jnp.where(jnp.logical_and(act, jnp.logical_not(big)), mid + 1, lo)
            hi2 = jnp.where(jnp.logical_and(act, big), mid, hi)
            return lo2, hi2
        lo, _ = lax.fori_loop(0, 18, b, (jnp.int32(0), jnp.int32(E)))
        return lo

    r0 = lower_bound(n0)
    r1 = jnp.where(n1 >= N, jnp.int32(E), lower_bound(n1))

    def emit(n_loc, d):
        f = lax.shift_right_logical(n_loc, 4)
        within = jnp.bitwise_and(n_loc, 15)
        parity = jnp.bitwise_and(f, 1)

        def wait_par():
            def w0():
                pltpu.make_async_copy(
                    stage_v.at[pl.ds(0, FLW)], out_hbm.at[pl.ds(0, FLW)], semF0
                ).wait()
            def w1():
                pltpu.make_async_copy(
                    stage_v.at[pl.ds(FLW, FLW)], out_hbm.at[pl.ds(0, FLW)], semF1
                ).wait()
            lax.cond(parity == 0, w0, w1)

        lax.cond(jnp.logical_and(within == 0, f >= 2), wait_par, lambda: None)

        st = pl.multiple_of(parity * FLW + within * OUTW, OUTW)
        for h in range(H):
            dh = bc(d, h)
            inv = 1.0 / jnp.where(dh > 0, dh, 1.0)
            for t in range(NT):
                stage_v[pl.ds(st + h * D + 16 * t, 16)] = _celu3(
                    o_buf[pl.ds(h * D + 16 * t, 16)] * inv)
        for i in range(H * NT):
            o_buf[pl.ds(16 * i, 16)] = zero16

        def start_flush():
            dst = pl.multiple_of((n0 + n_loc - (FLROWS - 1)) * OUTW, FLW)
            def s0():
                pltpu.async_copy(
                    stage_v.at[pl.ds(0, FLW)], out_hbm.at[pl.ds(dst, FLW)], semF0)
            def s1():
                pltpu.async_copy(
                    stage_v.at[pl.ds(FLW, FLW)], out_hbm.at[pl.ds(dst, FLW)], semF1)
            lax.cond(parity == 0, s0, s1)

        lax.cond(within == FLROWS - 1, start_flush, lambda: None)

    def emit_zero(n_loc):
        emit(n_loc, d0)

    BS0 = (r0 // BLK) * BLK
    nblocks = (r1 - BS0 + BLK - 1) // BLK

    def start_block(gi):
        bs = BS0 + gi * BLK
        slot = jnp.bitwise_and(gi, 1)
        def s0():
            pltpu.async_copy(emb_hbm.at[pl.ds(pl.multiple_of(bs * D, D), BLK * D)],
                             emb_v.at[pl.ds(0, BLK * D)], semE0)
            pltpu.async_copy(seg_hbm.at[pl.ds(pl.multiple_of(bs, 8), BLK)],
                             seg_v.at[pl.ds(0, BLK)], semS0)
        def s1():
            pltpu.async_copy(emb_hbm.at[pl.ds(pl.multiple_of(bs * D, D), BLK * D)],
                             emb_v.at[pl.ds(BLK * D, BLK * D)], semE1)
            pltpu.async_copy(seg_hbm.at[pl.ds(pl.multiple_of(bs, 8), BLK)],
                             seg_v.at[pl.ds(BLK, BLK)], semS1)
        lax.cond(slot == 0, s0, s1)

    def wait_block(parity):
        def w0():
            pltpu.make_async_copy(emb_hbm.at[pl.ds(0, BLK * D)],
                                  emb_v.at[pl.ds(0, BLK * D)], semE0).wait()
            pltpu.make_async_copy(seg_hbm.at[pl.ds(0, BLK)],
                                  seg_v.at[pl.ds(0, BLK)], semS0).wait()
        def w1():
            pltpu.make_async_copy(emb_hbm.at[pl.ds(0, BLK * D)],
                                  emb_v.at[pl.ds(BLK * D, BLK * D)], semE1).wait()
            pltpu.make_async_copy(seg_hbm.at[pl.ds(0, BLK)],
                                  seg_v.at[pl.ds(BLK, BLK)], semS1).wait()
        lax.cond(parity == 0, w0, w1)

    n_init = jnp.where(r0 < r1, seg_probe(jnp.minimum(r0, E - 1)), n0)

    lax.fori_loop(n0, n_init, lambda k, _: (emit_zero(k - n0), 0)[1], 0)

    lax.cond(nblocks > 0, lambda: start_block(jnp.int32(0)), lambda: None)

    def block_body(g, carry):
        p_in, n_in, m_in, dv_in = carry
        parity = jnp.bitwise_and(g, 1)
        wait_block(parity)
        lax.cond(g + 1 < nblocks, lambda: start_block(g + 1), lambda: None)
        bs = BS0 + g * BLK
        be = jnp.minimum(bs + BLK, r1)
        vbase = parity * (BLK * D)
        sbase = parity * BLK

        def seg_at(idx_local):
            grp = pl.multiple_of(sbase + (idx_local // 16) * 16, 16)
            return ext_i32(seg_v[pl.ds(grp, 16)], idx_local - (idx_local // 16) * 16)

        def walk_cond(st):
            return st[0] < be

        def walk_body(st):
            p, n, m, dv = st
            a1row = a1_v[pl.ds(pl.multiple_of((n - n0) * 16, 16), 16)]

            def scan_cond(s2):
                q, e = s2
                return jnp.logical_and(e < 0, q < be - bs)

            def scan_body(s2):
                q, _ = s2
                qa = pl.multiple_of(sbase + q, 16)
                v = seg_v[pl.ds(qa, 16)]
                inwin = jnp.logical_and(q + lanes >= p - bs, q + lanes < be - bs)
                vm = jnp.where(inwin, v, n)
                fs = jnp.max(plsc.all_reduce_ffs(vm > n))
                e2 = jnp.where(fs < 16, q + fs, -1)
                return q + 16, e2

            q0 = ((p - bs) // 16) * 16
            _, e_loc = lax.while_loop(scan_cond, scan_body, (q0, jnp.int32(-1)))
            e_abs = bs + jnp.where(e_loc >= 0, e_loc, be - bs)

            def inst_body(r, c3):
                m, dv = c3
                roff = pl.multiple_of(vbase + (r - bs) * D, 16)
                eft = [_celu3(emb_v[pl.ds(roff + 16 * t, 16)]) for t in range(NT)]
                a2vec = zero16
                for h in range(H):
                    acc = eft[0] * a2w[h * NT]
                    for t in range(1, NT):
                        acc = acc + eft[t] * a2w[h * NT + t]
                    a2vec = a2vec + jnp.sum(acc) * oneh[h]
                av = _celu3(a1row + a2vec)
                mn = jnp.maximum(m, av)
                w = jnp.exp(av - mn)

                def resc():
                    sc = jnp.exp(m - mn)
                    dvs = dv * sc
                    for h in range(H):
                        sch = bc(sc, h)
                        for t in range(NT):
                            sl = pl.ds(h * D + 16 * t, 16)
                            o_buf[sl] = o_buf[sl] * sch
                    return dvs

                changed = jnp.max(mn - m) > 0.0
                dv = lax.cond(changed, resc, lambda: dv) + w
                for h in range(H):
                    wh = bc(w, h)
                    for t in range(NT):
                        plsc.addupdate(o_buf.at[pl.ds(h * D + 16 * t, 16)],
                                       wh * eft[t])
                return mn, dv

            m, dv = lax.fori_loop(p, e_abs, inst_body, (m, dv))

            def fin():
                emit(n - n0, dv)
                nn = seg_at(e_abs - bs)
                lax.fori_loop(n + 1, nn,
                              lambda k, _: (emit_zero(k - n0), 0)[1], 0)
                return nn, m0, d0

            def keep():
                return n, m, dv

            n2, m2, dv2 = lax.cond(e_abs < be, fin, keep)
            return e_abs, n2, m2, dv2

        p_out, n_out, m_out, dv_out = lax.while_loop(
            walk_cond, walk_body, (jnp.maximum(p_in, bs), n_in, m_in, dv_in))
        return p_out, n_out, m_out, dv_out

    _, n_fin, _, dv_fin = lax.fori_loop(
        0, nblocks, block_body, (r0, n_init, m0, d0))

    emit(n_fin - n0, dv_fin)
    lax.fori_loop(n_fin + 1, n1, lambda k, _: (emit_zero(k - n0), 0)[1], 0)

    pltpu.make_async_copy(
        stage_v.at[pl.ds(0, FLW)], out_hbm.at[pl.ds(0, FLW)], semF0).wait()
    pltpu.make_async_copy(
        stage_v.at[pl.ds(FLW, FLW)], out_hbm.at[pl.ds(0, FLW)], semF1).wait()


def kernel(nodes, metapath_instances, metapath_embedding, features, attn1_w, attn2):
    w16 = jnp.zeros((D, 16), jnp.float32).at[:, :H].set(attn1_w.T)
    a1 = _a1_tc(features, w16)
    out = _sc_main(
        a1.reshape(-1),
        metapath_instances,
        metapath_embedding.reshape(-1),
        attn2.reshape(-1),
    )
    return out.reshape(N, OUTW)


# speedup vs baseline: 13.6607x; 1.7148x over previous
import functools

import jax
import jax.numpy as jnp
from jax import lax
from jax.experimental import pallas as pl
from jax.experimental.pallas import tpu as pltpu
from jax.experimental.pallas import tpu_sc as plsc

N = 10000
E = 160000
H = 4
D = 128
NT = D // 16

NC, NS = 2, 16
NW = NC * NS
NPS = 320
BLK = 128
EB = 2000
OUTW = H * D
FLROWS = 16
FLW = FLROWS * OUTW


def _celu3(x):
    return jnp.where(x > 0, x, 3.0 * jnp.exp(x * (1.0 / 3.0)) - 3.0)


def _a1_body(f_ref, w_ref, o_ref):
    x = jnp.dot(f_ref[...], w_ref[...], preferred_element_type=jnp.float32)
    o_ref[...] = _celu3(x)


def _a1_tc(features, w16):
    return pl.pallas_call(
        _a1_body,
        grid=(10,),
        in_specs=[
            pl.BlockSpec((N // 10, D), lambda i: (i, 0)),
            pl.BlockSpec((D, 16), lambda i: (0, 0)),
        ],
        out_specs=pl.BlockSpec((N // 10, 16), lambda i: (i, 0)),
        out_shape=jax.ShapeDtypeStruct((N, 16), jnp.float32),
    )(features, w16)


def _pre_body(e_ref, w_ref, eft_ref, a2_ref):
    eft = _celu3(e_ref[...])
    eft_ref[...] = eft
    a2_ref[...] = jnp.dot(eft, w_ref[...], preferred_element_type=jnp.float32)


def _pre_tc(emb, w16):
    return pl.pallas_call(
        _pre_body,
        grid=(E // EB,),
        in_specs=[
            pl.BlockSpec((EB, D), lambda i: (i, 0)),
            pl.BlockSpec((D, 16), lambda i: (0, 0)),
        ],
        out_specs=[
            pl.BlockSpec((EB, D), lambda i: (i, 0)),
            pl.BlockSpec((EB, 16), lambda i: (i, 0)),
        ],
        out_shape=[
            jax.ShapeDtypeStruct((E, D), jnp.float32),
            jax.ShapeDtypeStruct((E, 16), jnp.float32),
        ],
    )(emb, w16)


_MESH = plsc.VectorSubcoreMesh(
    core_axis_name="c", subcore_axis_name="s", num_cores=NC, num_subcores=NS
)


@functools.partial(
    pl.kernel,
    out_type=jax.ShapeDtypeStruct((N * OUTW,), jnp.float32),
    mesh=_MESH,
    compiler_params=pltpu.CompilerParams(needs_layout_passes=False),
    scratch_types=[
        pltpu.VMEM((NPS * 16,), jnp.float32),
        pltpu.VMEM((2 * BLK,), jnp.int32),
        pltpu.VMEM((2 * BLK * D,), jnp.float32),
        pltpu.VMEM((2 * BLK * 16,), jnp.float32),
        pltpu.VMEM((2 * FLW,), jnp.float32),
        pltpu.VMEM((16,), jnp.int32),
        pltpu.SemaphoreType.DMA,
        pltpu.SemaphoreType.DMA,
        pltpu.SemaphoreType.DMA,
        pltpu.SemaphoreType.DMA,
        pltpu.SemaphoreType.DMA,
        pltpu.SemaphoreType.DMA,
    ],
)
def _sc_main(a1_hbm, seg_hbm, eft_hbm, a2_hbm, out_hbm,
             a1_v, seg_v, eft_v, a2_v, stage_v, probe_v,
             semE0, semE1, semS0, semS1, semF0, semF1):
    wid = lax.axis_index("c") * NS + lax.axis_index("s")
    n0 = wid * NPS
    num = jnp.minimum(NPS, N - n0)
    n1 = n0 + num

    pltpu.sync_copy(a1_hbm.at[pl.ds(pl.multiple_of(n0 * 16, 16), NPS * 16)], a1_v)

    lanes = lax.iota(jnp.int32, 16)
    idxc = [jnp.full((16,), h, jnp.int32) for h in range(H)]
    _dn = lax.GatherDimensionNumbers(
        offset_dims=(), collapsed_slice_dims=(0,), start_index_map=(0,))
    zero16 = jnp.zeros((16,), jnp.float32)
    d0 = zero16
    o0 = tuple(zero16 for _ in range(H * NT))
    m0 = jnp.full((16,), -1e30, jnp.float32)

    def bc(v, h):
        return lax.gather(v, idxc[h][:, None], dimension_numbers=_dn,
                          slice_sizes=(1,),
                          mode=lax.GatherScatterMode.PROMISE_IN_BOUNDS)

    def ext_i32(vec, lane):
        return jnp.sum(jnp.where(lanes == lane, vec, 0))

    def seg_probe(pos):
        off = jnp.minimum(pos - lax.rem(pos, 8), E - 16)
        off = pl.multiple_of(off, 8)
        pltpu.sync_copy(seg_hbm.at[pl.ds(off, 16)], probe_v)
        return ext_i32(probe_v[pl.ds(0, 16)], pos - off)

    def lower_bound(tgt):
        def b(_, st):
            lo, hi = st
            mid = (lo + hi) // 2
            v = seg_probe(mid)
            act = lo < hi
            big = v >= tgt
            lo2 = jnp.where(jnp.logical_and(act, jnp.logical_not(big)), mid + 1, lo)
            hi2 = jnp.where(jnp.logical_and(act, big), mid, hi)
            return lo2, hi2
        lo, _ = lax.fori_loop(0, 18, b, (jnp.int32(0), jnp.int32(E)))
        return lo

    r0 = lower_bound(n0)
    r1 = jnp.where(n1 >= N, jnp.int32(E), lower_bound(n1))

    def emit(n_loc, d, o):
        f = lax.shift_right_logical(n_loc, 4)
        within = jnp.bitwise_and(n_loc, 15)
        parity = jnp.bitwise_and(f, 1)

        def wait_par():
            def w0():
                pltpu.make_async_copy(
                    stage_v.at[pl.ds(0, FLW)], out_hbm.at[pl.ds(0, FLW)], semF0
                ).wait()
            def w1():
                pltpu.make_async_copy(
                    stage_v.at[pl.ds(FLW, FLW)], out_hbm.at[pl.ds(0, FLW)], semF1
                ).wait()
            lax.cond(parity == 0, w0, w1)

        lax.cond(jnp.logical_and(within == 0, f >= 2), wait_par, lambda: None)

        st = pl.multiple_of(parity * FLW + within * OUTW, OUTW)
        for h in range(H):
            dh = bc(d, h)
            inv = 1.0 / jnp.where(dh > 0, dh, 1.0)
            for t in range(NT):
                stage_v[pl.ds(st + h * D + 16 * t, 16)] = _celu3(
                    o[h * NT + t] * inv)

        def start_flush():
            dst = pl.multiple_of((n0 + n_loc - (FLROWS - 1)) * OUTW, FLW)
            def s0():
                pltpu.async_copy(
                    stage_v.at[pl.ds(0, FLW)], out_hbm.at[pl.ds(dst, FLW)], semF0)
            def s1():
                pltpu.async_copy(
                    stage_v.at[pl.ds(FLW, FLW)], out_hbm.at[pl.ds(dst, FLW)], semF1)
            lax.cond(parity == 0, s0, s1)

        lax.cond(within == FLROWS - 1, start_flush, lambda: None)

    def emit_zero(n_loc):
        emit(n_loc, d0, o0)

    BS0 = (r0 // BLK) * BLK
    nblocks = (r1 - BS0 + BLK - 1) // BLK

    def start_block(gi):
        bs = BS0 + gi * BLK
        slot = jnp.bitwise_and(gi, 1)
        def s0():
            pltpu.async_copy(eft_hbm.at[pl.ds(pl.multiple_of(bs * D, D), BLK * D)],
                             eft_v.at[pl.ds(0, BLK * D)], semE0)
            pltpu.async_copy(seg_hbm.at[pl.ds(pl.multiple_of(bs, 8), BLK)],
                             seg_v.at[pl.ds(0, BLK)], semS0)
            pltpu.async_copy(a2_hbm.at[pl.ds(pl.multiple_of(bs * 16, 16), BLK * 16)],
                             a2_v.at[pl.ds(0, BLK * 16)], semS0)
        def s1():
            pltpu.async_copy(eft_hbm.at[pl.ds(pl.multiple_of(bs * D, D), BLK * D)],
                             eft_v.at[pl.ds(BLK * D, BLK * D)], semE1)
            pltpu.async_copy(seg_hbm.at[pl.ds(pl.multiple_of(bs, 8), BLK)],
                             seg_v.at[pl.ds(BLK, BLK)], semS1)
            pltpu.async_copy(a2_hbm.at[pl.ds(pl.multiple_of(bs * 16, 16), BLK * 16)],
                             a2_v.at[pl.ds(BLK * 16, BLK * 16)], semS1)
        lax.cond(slot == 0, s0, s1)

    def wait_block(parity):
        def w0():
            pltpu.make_async_copy(eft_hbm.at[pl.ds(0, BLK * D)],
                                  eft_v.at[pl.ds(0, BLK * D)], semE0).wait()
            pltpu.make_async_copy(seg_hbm.at[pl.ds(0, BLK)],
                                  seg_v.at[pl.ds(0, BLK)], semS0).wait()
            pltpu.make_async_copy(a2_hbm.at[pl.ds(0, BLK * 16)],
                                  a2_v.at[pl.ds(0, BLK * 16)], semS0).wait()
        def w1():
            pltpu.make_async_copy(eft_hbm.at[pl.ds(0, BLK * D)],
                                  eft_v.at[pl.ds(BLK * D, BLK * D)], semE1).wait()
            pltpu.make_async_copy(seg_hbm.at[pl.ds(0, BLK)],
                                  seg_v.at[pl.ds(BLK, BLK)], semS1).wait()
            pltpu.make_async_copy(a2_hbm.at[pl.ds(0, BLK * 16)],
                                  a2_v.at[pl.ds(BLK * 16, BLK * 16)], semS1).wait()
        lax.cond(parity == 0, w0, w1)

    n_init = jnp.where(r0 < r1, seg_probe(jnp.minimum(r0, E - 1)), n0)

    lax.fori_loop(n0, n_init, lambda k, _: (emit_zero(k - n0), 0)[1], 0)

    lax.cond(nblocks > 0, lambda: start_block(jnp.int32(0)), lambda: None)

    def block_body(g, carry):
        p_in, n_in, m_in, dv_in, o_in = carry
        parity = jnp.bitwise_and(g, 1)
        wait_block(parity)
        lax.cond(g + 1 < nblocks, lambda: start_block(g + 1), lambda: None)
        bs = BS0 + g * BLK
        be = jnp.minimum(bs + BLK, r1)
        vbase = parity * (BLK * D)
        abase = parity * (BLK * 16)
        sbase = parity * BLK

        def seg_at(idx_local):
            grp = pl.multiple_of(sbase + (idx_local // 16) * 16, 16)
            return ext_i32(seg_v[pl.ds(grp, 16)], idx_local - (idx_local // 16) * 16)

        def walk_cond(st):
            return st[0] < be

        def walk_body(st):
            p, n, m, dv, o = st
            a1row = a1_v[pl.ds(pl.multiple_of((n - n0) * 16, 16), 16)]

            def scan_cond(s2):
                q, e = s2
                return jnp.logical_and(e < 0, q < be - bs)

            def scan_body(s2):
                q, _ = s2
                qa = pl.multiple_of(sbase + q, 16)
                v = seg_v[pl.ds(qa, 16)]
                inwin = jnp.logical_and(q + lanes >= p - bs, q + lanes < be - bs)
                vm = jnp.where(inwin, v, n)
                fs = jnp.max(plsc.all_reduce_ffs(vm > n))
                e2 = jnp.where(fs < 16, q + fs, -1)
                return q + 16, e2

            q0 = ((p - bs) // 16) * 16
            _, e_loc = lax.while_loop(scan_cond, scan_body, (q0, jnp.int32(-1)))
            e_abs = bs + jnp.where(e_loc >= 0, e_loc, be - bs)

            def inst_body(r, c3):
                m, dv, o = c3
                roff = pl.multiple_of(vbase + (r - bs) * D, 16)
                eft = [eft_v[pl.ds(roff + 16 * t, 16)] for t in range(NT)]
                a2row = a2_v[pl.ds(pl.multiple_of(abase + (r - bs) * 16, 16), 16)]
                av = _celu3(a1row + a2row)
                mn = jnp.maximum(m, av)
                sc = jnp.exp(m - mn)
                w = jnp.exp(av - mn)
                dv = dv * sc + w
                o = list(o)
                for h in range(H):
                    wh = bc(w, h)
                    sch = bc(sc, h)
                    for t in range(NT):
                        o[h * NT + t] = o[h * NT + t] * sch + wh * eft[t]
                return mn, dv, tuple(o)

            m, dv, o = lax.fori_loop(p, e_abs, inst_body, (m, dv, o))

            def fin():
                emit(n - n0, dv, o)
                nn = seg_at(e_abs - bs)
                lax.fori_loop(n + 1, nn,
                              lambda k, _: (emit_zero(k - n0), 0)[1], 0)
                return nn, m0, d0, o0

            def keep():
                return n, m, dv, o

            n2, m2, dv2, o2 = lax.cond(e_abs < be, fin, keep)
            return e_abs, n2, m2, dv2, o2

        p_out, n_out, m_out, dv_out, o_out = lax.while_loop(
            walk_cond, walk_body, (jnp.maximum(p_in, bs), n_in, m_in, dv_in, o_in))
        return p_out, n_out, m_out, dv_out, o_out

    _, n_fin, _, dv_fin, o_fin = lax.fori_loop(
        0, nblocks, block_body, (r0, n_init, m0, d0, o0))

    emit(n_fin - n0, dv_fin, o_fin)
    lax.fori_loop(n_fin + 1, n1, lambda k, _: (emit_zero(k - n0), 0)[1], 0)

    pltpu.make_async_copy(
        stage_v.at[pl.ds(0, FLW)], out_hbm.at[pl.ds(0, FLW)], semF0).wait()
    pltpu.make_async_copy(
        stage_v.at[pl.ds(FLW, FLW)], out_hbm.at[pl.ds(0, FLW)], semF1).wait()


def kernel(nodes, metapath_instances, metapath_embedding, features, attn1_w, attn2):
    w16 = jnp.zeros((D, 16), jnp.float32).at[:, :H].set(attn1_w.T)
    a1 = _a1_tc(features, w16)
    aw16 = jnp.zeros((D, 16), jnp.float32).at[:, :H].set(attn2.reshape(H, D).T)
    eft, a2p = _pre_tc(metapath_embedding, aw16)
    out = _sc_main(
        a1.reshape(-1),
        metapath_instances,
        eft.reshape(-1),
        a2p.reshape(-1),
    )
    return out.reshape(N, OUTW)
